# baseline (speedup 1.0000x reference)
import functools

import jax
import jax.numpy as jnp
from jax import lax
from jax.experimental import pallas as pl
from jax.experimental.pallas import tpu as pltpu
from jax.experimental.pallas import tpu_sc as plsc

_N = 10000
_NPAD = 10112
_ROWS_PER_SUB = _NPAD // 16
_CHUNK = 128
_NSC = 2
_NSUB = 16
_NW = _NSC * _NSUB


def _sc_mesh():
    return plsc.VectorSubcoreMesh(core_axis_name="c", subcore_axis_name="s")


def _tile_sched(c, s, cpt_lo, n_hi):
    wid = c * _NSUB + s
    cpt = jnp.where(wid < n_hi, cpt_lo + 2, cpt_lo)
    ebase = cpt_lo * wid + 2 * jnp.minimum(wid, n_hi)
    return cpt, ebase


def _make_deg_kernel(cpt_lo, n_hi):
    F = 16
    cpt_max = cpt_lo + (2 if n_hi else 0)

    @functools.partial(
        pl.kernel,
        mesh=_sc_mesh(),
        out_type=jax.ShapeDtypeStruct((_NSC, _NPAD, 8), jnp.float32),
        scratch_types=[
            pltpu.VMEM((cpt_max, _CHUNK), jnp.int32),
            pltpu.VMEM((_CHUNK, F), jnp.float32),
            pltpu.VMEM_SHARED((_NPAD, F), jnp.float32),
            pltpu.SemaphoreType.DMA,
        ],
        compiler_params=pltpu.CompilerParams(use_tc_tiling_on_sc=False),
    )
    def deg_kernel(dst_hbm, ones_hbm, zeros_hbm, out_hbm, dst_v, ones_v, acc_sh, sem):
        c = lax.axis_index("c")
        s = lax.axis_index("s")
        cpt, ebase = _tile_sched(c, s, cpt_lo, n_hi)
        pltpu.sync_copy(dst_hbm.at[pl.ds(ebase, cpt_max)], dst_v)
        pltpu.sync_copy(ones_hbm, ones_v)
        nbase = s * _ROWS_PER_SUB
        pltpu.sync_copy(zeros_hbm.at[pl.ds(nbase, _ROWS_PER_SUB)],
                        acc_sh.at[pl.ds(nbase, _ROWS_PER_SUB)])
        plsc.subcore_barrier()

        def body(j, carry):
            pltpu.async_copy(ones_v, acc_sh.at[dst_v.at[j]], sem, add=True)
            return carry

        lax.fori_loop(0, cpt, body, 0)

        def drain(j, carry):
            pltpu.make_async_copy(ones_v, acc_sh.at[dst_v.at[j]], sem).wait()
            return carry

        lax.fori_loop(0, cpt, drain, 0)
        plsc.subcore_barrier()
        pltpu.sync_copy(acc_sh.at[pl.ds(nbase, _ROWS_PER_SUB), pl.ds(0, 8)],
                        out_hbm.at[c, pl.ds(nbase, _ROWS_PER_SUB)])

    return deg_kernel


def _make_scatter_kernel(F, cpt_lo, n_hi):
    cpt_max = cpt_lo + (2 if n_hi else 0)

    @functools.partial(
        pl.kernel,
        mesh=_sc_mesh(),
        out_type=jax.ShapeDtypeStruct((_NSC, _NPAD, F), jnp.float32),
        scratch_types=[
            pltpu.VMEM((cpt_max, _CHUNK), jnp.int32),
            pltpu.VMEM((cpt_max, _CHUNK), jnp.int32),
            pltpu.VMEM((2, _CHUNK, F), jnp.float32),
            pltpu.VMEM_SHARED((_NPAD, F), jnp.float32),
            pltpu.VMEM_SHARED((_NPAD, F), jnp.float32),
        ] + [pltpu.SemaphoreType.DMA] * 2,
        compiler_params=pltpu.CompilerParams(use_tc_tiling_on_sc=False),
    )
    def scatter_kernel(g_hbm, src_hbm, dst_hbm, zeros_hbm, out_hbm,
                       src_v, dst_v, rows_v, acc_sh, g_sh, *sems):
        c = lax.axis_index("c")
        s = lax.axis_index("s")
        cpt, ebase = _tile_sched(c, s, cpt_lo, n_hi)
        pltpu.sync_copy(src_hbm.at[pl.ds(ebase, cpt_max)], src_v)
        pltpu.sync_copy(dst_hbm.at[pl.ds(ebase, cpt_max)], dst_v)
        nbase = s * _ROWS_PER_SUB
        pltpu.sync_copy(zeros_hbm.at[pl.ds(nbase, _ROWS_PER_SUB)],
                        acc_sh.at[pl.ds(nbase, _ROWS_PER_SUB)])
        pltpu.sync_copy(g_hbm.at[pl.ds(nbase, _ROWS_PER_SUB)],
                        g_sh.at[pl.ds(nbase, _ROWS_PER_SUB)])
        plsc.subcore_barrier()

        def gather_start(j, b):
            pltpu.async_copy(g_sh.at[src_v.at[j]], rows_v.at[b], sems[b])

        def gather_wait(j, b):
            pltpu.make_async_copy(g_sh.at[src_v.at[j]], rows_v.at[b], sems[b]).wait()

        def scat(j, b):
            pltpu.sync_copy(rows_v.at[b], acc_sh.at[dst_v.at[j]], add=True)

        gather_start(0, 0)
        gather_start(1, 1)

        def body(j0, carry):
            j = j0 * 2
            for b in range(2):
                gather_wait(j + b, b)
                scat(j + b, b)
                gather_start(j + b + 2, b)
            return carry

        lax.fori_loop(0, (cpt - 2) // 2, body, 0)
        for b in range(2):
            j = cpt - 2 + b
            gather_wait(j, b)
            scat(j, b)
        plsc.subcore_barrier()
        pltpu.sync_copy(acc_sh.at[pl.ds(nbase, _ROWS_PER_SUB)],
                        out_hbm.at[c, pl.ds(nbase, _ROWS_PER_SUB)])

    return scatter_kernel


def _tc1a_body(x_ref, w1_ref, xw1_ref):
    xw1_ref[pl.ds(0, _N), :] = jnp.dot(x_ref[...], w1_ref[...],
                                       preferred_element_type=jnp.float32)
    xw1_ref[pl.ds(_N, _NPAD - _N), :] = jnp.zeros((_NPAD - _N, 64), jnp.float32)


def _tc1b_body(xw1_ref, degp_ref, g1_ref, dis_ref):
    deg = degp_ref[0, :, 0:1] + degp_ref[1, :, 0:1] + 1.0
    dis = lax.rsqrt(deg)
    dis_ref[...] = dis
    g1_ref[...] = xw1_ref[...] * dis


def _tc1c_body(x_ref, ws02_ref, ws03_ref, xs02_ref, xs03_ref):
    x = x_ref[...]
    pad = _NPAD - _N
    xs02_ref[pl.ds(0, _N), :] = jnp.dot(x, ws02_ref[...],
                                        preferred_element_type=jnp.float32)
    xs02_ref[pl.ds(_N, pad), :] = jnp.zeros((pad, 32), jnp.float32)
    xs03_ref[pl.ds(0, _N), :] = jnp.dot(x, ws03_ref[...],
                                        preferred_element_type=jnp.float32)
    xs03_ref[pl.ds(_N, pad), :] = jnp.zeros((pad, 16), jnp.float32)


def _tc2_body(s1_ref, g1_ref, dis_ref, b1_ref, w2_ref,
              g2_ref, x1_ref):
    dis = dis_ref[...]
    x1 = jnp.maximum((s1_ref[0] + s1_ref[1] + g1_ref[...]) * dis + b1_ref[...], 0.0)
    x1_ref[...] = x1
    g2_ref[...] = jnp.dot(x1, w2_ref[...], preferred_element_type=jnp.float32) * dis


def _tc2b_body(x1_ref, ws13_ref, xs13_ref):
    xs13_ref[...] = jnp.dot(x1_ref[...], ws13_ref[...], preferred_element_type=jnp.float32)


def _tc3_body(s2_ref, g2_ref, xs02_ref, dis_ref, b2_ref, bs02_ref, w3_ref,
              g3_ref):
    dis = dis_ref[...]
    x2 = jnp.maximum((s2_ref[0] + s2_ref[1] + g2_ref[...]) * dis + b2_ref[...]
                     + xs02_ref[...] + bs02_ref[...], 0.0)
    g3_ref[...] = jnp.dot(x2, w3_ref[...], preferred_element_type=jnp.float32) * dis


def _tc4_body(s3_ref, g3_ref, xs03_ref, xs13_ref, dis_ref,
              b3_ref, bs03_ref, bs13_ref, wl_ref, bl_ref, out_ref):
    dis = dis_ref[...]
    x3 = jnp.maximum((s3_ref[0] + s3_ref[1] + g3_ref[...]) * dis + b3_ref[...]
                     + xs03_ref[...] + bs03_ref[...] + xs13_ref[...] + bs13_ref[...], 0.0)
    z = jnp.dot(x3, wl_ref[...], preferred_element_type=jnp.float32) + bl_ref[...]
    out_ref[...] = jax.nn.sigmoid(z)


def _f32(shape):
    return jax.ShapeDtypeStruct(shape, jnp.float32)


def kernel(x, edge_index, W1, b1, Ws02, bs02, W2, b2, Ws03, bs03, Ws13, bs13, W3, b3, Wl, bl):
    E = edge_index.shape[1]
    nchunks = 2 * ((E + 2 * _CHUNK - 1) // (2 * _CHUNK))
    npairs = nchunks // 2
    cpt_lo = 2 * (npairs // _NW)
    n_hi = npairs - _NW * (npairs // _NW)
    sched = (cpt_lo, n_hi)

    ep = (nchunks + 2) * _CHUNK
    pad = ep - E
    fill = jnp.full((pad,), _N, jnp.int32)
    src = jnp.concatenate([edge_index[0], fill]).reshape(ep // _CHUNK, _CHUNK)
    dst = jnp.concatenate([edge_index[1], fill]).reshape(ep // _CHUNK, _CHUNK)
    ones16 = jnp.ones((_CHUNK, 16), jnp.float32)
    z64 = jnp.zeros((_NPAD, 64), jnp.float32)
    z32 = jnp.zeros((_NPAD, 32), jnp.float32)
    z16 = jnp.zeros((_NPAD, 16), jnp.float32)

    degp = _make_deg_kernel(*sched)(dst, ones16, z16)

    xw1 = pl.pallas_call(_tc1a_body, out_shape=_f32((_NPAD, 64)))(x, W1)
    g1, dis = pl.pallas_call(
        _tc1b_body,
        out_shape=[_f32((_NPAD, 64)), _f32((_NPAD, 1))],
    )(xw1, degp)

    s1 = _make_scatter_kernel(64, *sched)(g1, src, dst, z64)

    xs02, xs03 = pl.pallas_call(
        _tc1c_body,
        out_shape=[_f32((_NPAD, 32)), _f32((_NPAD, 16))],
    )(x, Ws02, Ws03)

    g2, x1 = pl.pallas_call(
        _tc2_body,
        out_shape=[_f32((_NPAD, 32)), _f32((_NPAD, 64))],
    )(s1, g1, dis, b1.reshape(1, -1), W2)

    s2 = _make_scatter_kernel(32, *sched)(g2, src, dst, z32)

    xs13 = pl.pallas_call(_tc2b_body, out_shape=_f32((_NPAD, 16)))(x1, Ws13)

    g3 = pl.pallas_call(
        _tc3_body,
        out_shape=_f32((_NPAD, 16)),
    )(s2, g2, xs02, dis, b2.reshape(1, -1), bs02.reshape(1, -1), W3)

    s3 = _make_scatter_kernel(16, *sched)(g3, src, dst, z16)

    out = pl.pallas_call(
        _tc4_body,
        out_shape=_f32((_NPAD, 1)),
    )(s3, g3, xs03, xs13, dis, b3.reshape(1, -1), bs03.reshape(1, -1),
      bs13.reshape(1, -1), Wl, bl.reshape(1, -1))

    return out[:_N]


# speedup vs baseline: 1.0222x; 1.0222x over previous
import functools

import jax
import jax.numpy as jnp
from jax import lax
from jax.experimental import pallas as pl
from jax.experimental.pallas import tpu as pltpu
from jax.experimental.pallas import tpu_sc as plsc

_N = 10000
_NPAD = 10112
_ROWS_PER_SUB = _NPAD // 16
_CHUNK = 128
_NSC = 2
_NSUB = 16
_NW = _NSC * _NSUB


def _sc_mesh():
    return plsc.VectorSubcoreMesh(core_axis_name="c", subcore_axis_name="s")


def _tile_sched(c, s, cpt_lo, n_hi):
    wid = c * _NSUB + s
    cpt = jnp.where(wid < n_hi, cpt_lo + 2, cpt_lo)
    ebase = cpt_lo * wid + 2 * jnp.minimum(wid, n_hi)
    return cpt, ebase


def _make_deg_kernel(cpt_lo, n_hi):
    F = 16
    cpt_max = cpt_lo + (2 if n_hi else 0)

    @functools.partial(
        pl.kernel,
        mesh=_sc_mesh(),
        out_type=jax.ShapeDtypeStruct((_NSC, _NPAD, F), jnp.float32),
        scratch_types=[
            pltpu.VMEM((cpt_max, _CHUNK), jnp.int32),
            pltpu.VMEM((_CHUNK, F), jnp.float32),
            pltpu.VMEM_SHARED((_NPAD, F), jnp.float32),
            pltpu.SemaphoreType.DMA,
        ],
        compiler_params=pltpu.CompilerParams(use_tc_tiling_on_sc=False),
    )
    def deg_kernel(dst_hbm, ones_hbm, zeros_hbm, out_hbm, dst_v, ones_v, acc_sh, sem):
        c = lax.axis_index("c")
        s = lax.axis_index("s")
        cpt, ebase = _tile_sched(c, s, cpt_lo, n_hi)
        pltpu.sync_copy(dst_hbm.at[pl.ds(ebase, cpt_max)], dst_v)
        pltpu.sync_copy(ones_hbm, ones_v)
        nbase = s * _ROWS_PER_SUB
        pltpu.sync_copy(zeros_hbm.at[pl.ds(nbase, _ROWS_PER_SUB)],
                        acc_sh.at[pl.ds(nbase, _ROWS_PER_SUB)])
        plsc.subcore_barrier()

        def body(j, carry):
            pltpu.async_copy(ones_v, acc_sh.at[dst_v.at[j]], sem, add=True)
            return carry

        lax.fori_loop(0, cpt, body, 0)

        def drain(j, carry):
            pltpu.make_async_copy(ones_v, acc_sh.at[dst_v.at[j]], sem).wait()
            return carry

        lax.fori_loop(0, cpt, drain, 0)
        plsc.subcore_barrier()
        pltpu.sync_copy(acc_sh.at[pl.ds(nbase, _ROWS_PER_SUB)],
                        out_hbm.at[c, pl.ds(nbase, _ROWS_PER_SUB)])

    return deg_kernel


def _make_scatter_kernel(F, cpt_lo, n_hi):
    cpt_max = cpt_lo + (2 if n_hi else 0)

    @functools.partial(
        pl.kernel,
        mesh=_sc_mesh(),
        out_type=jax.ShapeDtypeStruct((_NSC, _NPAD, F), jnp.float32),
        scratch_types=[
            pltpu.VMEM((cpt_max, _CHUNK), jnp.int32),
            pltpu.VMEM((cpt_max, _CHUNK), jnp.int32),
            pltpu.VMEM((2, _CHUNK, F), jnp.float32),
            pltpu.VMEM_SHARED((_NPAD, F), jnp.float32),
            pltpu.VMEM_SHARED((_NPAD, F), jnp.float32),
        ] + [pltpu.SemaphoreType.DMA] * 2,
        compiler_params=pltpu.CompilerParams(use_tc_tiling_on_sc=False),
    )
    def scatter_kernel(g_hbm, src_hbm, dst_hbm, zeros_hbm, out_hbm,
                       src_v, dst_v, rows_v, acc_sh, g_sh, *sems):
        c = lax.axis_index("c")
        s = lax.axis_index("s")
        cpt, ebase = _tile_sched(c, s, cpt_lo, n_hi)
        pltpu.sync_copy(src_hbm.at[pl.ds(ebase, cpt_max)], src_v)
        pltpu.sync_copy(dst_hbm.at[pl.ds(ebase, cpt_max)], dst_v)
        nbase = s * _ROWS_PER_SUB
        pltpu.sync_copy(zeros_hbm.at[pl.ds(nbase, _ROWS_PER_SUB)],
                        acc_sh.at[pl.ds(nbase, _ROWS_PER_SUB)])
        pltpu.sync_copy(g_hbm.at[pl.ds(nbase, _ROWS_PER_SUB)],
                        g_sh.at[pl.ds(nbase, _ROWS_PER_SUB)])
        plsc.subcore_barrier()

        def gather_start(j, b):
            pltpu.async_copy(g_sh.at[src_v.at[j]], rows_v.at[b], sems[b])

        def gather_wait(j, b):
            pltpu.make_async_copy(g_sh.at[src_v.at[j]], rows_v.at[b], sems[b]).wait()

        def scat(j, b):
            pltpu.sync_copy(rows_v.at[b], acc_sh.at[dst_v.at[j]], add=True)

        gather_start(0, 0)
        gather_start(1, 1)

        def body(j0, carry):
            j = j0 * 2
            for b in range(2):
                gather_wait(j + b, b)
                scat(j + b, b)
                gather_start(j + b + 2, b)
            return carry

        lax.fori_loop(0, (cpt - 2) // 2, body, 0)
        for b in range(2):
            j = cpt - 2 + b
            gather_wait(j, b)
            scat(j, b)
        plsc.subcore_barrier()
        pltpu.sync_copy(acc_sh.at[pl.ds(nbase, _ROWS_PER_SUB)],
                        out_hbm.at[c, pl.ds(nbase, _ROWS_PER_SUB)])

    return scatter_kernel


def _tc1a_body(x_ref, w1_ref, xw1_ref):
    xw1_ref[pl.ds(0, _N), :] = jnp.dot(x_ref[...], w1_ref[...],
                                       preferred_element_type=jnp.float32)
    xw1_ref[pl.ds(_N, _NPAD - _N), :] = jnp.zeros((_NPAD - _N, 64), jnp.float32)


def _tc1b_body(xw1_ref, degp_ref, g1_ref, dis_ref):
    deg = degp_ref[0, :, 0:1] + degp_ref[1, :, 0:1] + 1.0
    dis = lax.rsqrt(deg)
    dis_ref[...] = dis
    g1_ref[...] = xw1_ref[...] * dis


def _tc1c_body(x_ref, ws02_ref, ws03_ref, xs02_ref, xs03_ref):
    x = x_ref[...]
    pad = _NPAD - _N
    xs02_ref[pl.ds(0, _N), :] = jnp.dot(x, ws02_ref[...],
                                        preferred_element_type=jnp.float32)
    xs02_ref[pl.ds(_N, pad), :] = jnp.zeros((pad, 32), jnp.float32)
    xs03_ref[pl.ds(0, _N), :] = jnp.dot(x, ws03_ref[...],
                                        preferred_element_type=jnp.float32)
    xs03_ref[pl.ds(_N, pad), :] = jnp.zeros((pad, 16), jnp.float32)


def _tc2_body(s1_ref, g1_ref, dis_ref, b1_ref, w2_ref,
              g2_ref, x1_ref):
    dis = dis_ref[...]
    x1 = jnp.maximum((s1_ref[0] + s1_ref[1] + g1_ref[...]) * dis + b1_ref[...], 0.0)
    x1_ref[...] = x1
    g2_ref[...] = jnp.dot(x1, w2_ref[...], preferred_element_type=jnp.float32) * dis


def _tc2b_body(x1_ref, ws13_ref, xs13_ref):
    xs13_ref[...] = jnp.dot(x1_ref[...], ws13_ref[...], preferred_element_type=jnp.float32)


def _tc3_body(s2_ref, g2_ref, xs02_ref, dis_ref, b2_ref, bs02_ref, w3_ref,
              g3_ref):
    dis = dis_ref[...]
    x2 = jnp.maximum((s2_ref[0] + s2_ref[1] + g2_ref[...]) * dis + b2_ref[...]
                     + xs02_ref[...] + bs02_ref[...], 0.0)
    g3_ref[...] = jnp.dot(x2, w3_ref[...], preferred_element_type=jnp.float32) * dis


def _tc4_body(s3_ref, g3_ref, xs03_ref, xs13_ref, dis_ref,
              b3_ref, bs03_ref, bs13_ref, wl_ref, bl_ref, out_ref):
    dis = dis_ref[...]
    x3 = jnp.maximum((s3_ref[0] + s3_ref[1] + g3_ref[...]) * dis + b3_ref[...]
                     + xs03_ref[...] + bs03_ref[...] + xs13_ref[...] + bs13_ref[...], 0.0)
    z = jnp.dot(x3, wl_ref[...], preferred_element_type=jnp.float32) + bl_ref[...]
    out_ref[...] = jax.nn.sigmoid(z)


def _f32(shape):
    return jax.ShapeDtypeStruct(shape, jnp.float32)


def kernel(x, edge_index, W1, b1, Ws02, bs02, W2, b2, Ws03, bs03, Ws13, bs13, W3, b3, Wl, bl):
    E = edge_index.shape[1]
    nchunks = 2 * ((E + 2 * _CHUNK - 1) // (2 * _CHUNK))
    npairs = nchunks // 2
    cpt_lo = 2 * (npairs // _NW)
    n_hi = npairs - _NW * (npairs // _NW)
    sched = (cpt_lo, n_hi)

    ep = (nchunks + 2) * _CHUNK
    pad = ep - E
    fill = jnp.full((pad,), _N, jnp.int32)
    src = jnp.concatenate([edge_index[0], fill]).reshape(ep // _CHUNK, _CHUNK)
    dst = jnp.concatenate([edge_index[1], fill]).reshape(ep // _CHUNK, _CHUNK)
    ones16 = jnp.ones((_CHUNK, 16), jnp.float32)
    z64 = jnp.zeros((_NPAD, 64), jnp.float32)
    z32 = jnp.zeros((_NPAD, 32), jnp.float32)
    z16 = jnp.zeros((_NPAD, 16), jnp.float32)

    degp = _make_deg_kernel(*sched)(dst, ones16, z16)

    xw1 = pl.pallas_call(_tc1a_body, out_shape=_f32((_NPAD, 64)))(x, W1)
    g1, dis = pl.pallas_call(
        _tc1b_body,
        out_shape=[_f32((_NPAD, 64)), _f32((_NPAD, 1))],
    )(xw1, degp)

    s1 = _make_scatter_kernel(64, *sched)(g1, src, dst, z64)

    xs02, xs03 = pl.pallas_call(
        _tc1c_body,
        out_shape=[_f32((_NPAD, 32)), _f32((_NPAD, 16))],
    )(x, Ws02, Ws03)

    g2, x1 = pl.pallas_call(
        _tc2_body,
        out_shape=[_f32((_NPAD, 32)), _f32((_NPAD, 64))],
    )(s1, g1, dis, b1.reshape(1, -1), W2)

    s2 = _make_scatter_kernel(32, *sched)(g2, src, dst, z32)

    xs13 = pl.pallas_call(_tc2b_body, out_shape=_f32((_NPAD, 16)))(x1, Ws13)

    g3 = pl.pallas_call(
        _tc3_body,
        out_shape=_f32((_NPAD, 16)),
    )(s2, g2, xs02, dis, b2.reshape(1, -1), bs02.reshape(1, -1), W3)

    s3 = _make_scatter_kernel(16, *sched)(g3, src, dst, z16)

    out = pl.pallas_call(
        _tc4_body,
        out_shape=_f32((_NPAD, 1)),
    )(s3, g3, xs03, xs13, dis, b3.reshape(1, -1), bs03.reshape(1, -1),
      bs13.reshape(1, -1), Wl, bl.reshape(1, -1))

    return out[:_N]
